# Initial kernel scaffold; baseline (speedup 1.0000x reference)
import jax
import jax.numpy as jnp
from jax import lax
from jax.experimental import pallas as pl
from jax.experimental.pallas import tpu as pltpu
from jax.experimental.pallas import tpu_sc as plsc

N = 10000
E = 320000
D = 128

NC = 2
NS = 16
NW = NC * NS
CH = 128
EPW = 10240
K = EPW // CH
E_PAD = NW * EPW
NPAD = 10240
RPT = NPAD // NS
DUMMY = N


def _make_segsum(with_cnt):
  mesh = plsc.VectorSubcoreMesh(
      core_axis_name="c", subcore_axis_name="s", num_cores=NC,
      num_subcores=NS)

  out_type = [jax.ShapeDtypeStruct((NC, NPAD, D), jnp.float32)]
  scratch = [
      pltpu.VMEM((K, CH), jnp.int32),
      pltpu.VMEM((K, CH), jnp.int32),
      pltpu.VMEM((CH, D), jnp.float32),
      pltpu.VMEM_SHARED((NPAD, D), jnp.float32),
      pltpu.SemaphoreType.DMA,
  ]
  if with_cnt:
    out_type.append(jax.ShapeDtypeStruct((NC, NPAD, 16), jnp.float32))
    scratch += [
        pltpu.VMEM((CH, 16), jnp.float32),
        pltpu.VMEM_SHARED((NPAD, 16), jnp.float32),
    ]

  def body(x_hbm, src_hbm, dst_hbm, zf_hbm, zc_hbm, on_hbm, *rest):
    if with_cnt:
      (agg_out, cnt_out, src_v, dst_v, rows_v, acc_sh, sem, ones_v,
       cnt_sh) = rest
    else:
      agg_out, src_v, dst_v, rows_v, acc_sh, sem = rest
    c = lax.axis_index("c")
    s = lax.axis_index("s")
    w = s * NC + c

    pltpu.sync_copy(src_hbm.at[w], src_v)
    pltpu.sync_copy(dst_hbm.at[w], dst_v)
    if with_cnt:
      pltpu.sync_copy(on_hbm, ones_v)
    for r in range(RPT // CH):
      pltpu.sync_copy(zf_hbm, acc_sh.at[pl.ds(s * RPT + r * CH, CH)])
      if with_cnt:
        pltpu.sync_copy(zc_hbm, cnt_sh.at[pl.ds(s * RPT + r * CH, CH)])
    plsc.subcore_barrier()

    def step(j, carry):
      pltpu.async_copy(x_hbm.at[src_v.at[j]], rows_v, sem).wait()
      pltpu.sync_copy(rows_v, acc_sh.at[dst_v.at[j]], add=True)
      if with_cnt:
        pltpu.sync_copy(ones_v, cnt_sh.at[dst_v.at[j]], add=True)
      return carry

    lax.fori_loop(0, K, step, 0)
    plsc.subcore_barrier()

    pltpu.sync_copy(acc_sh.at[pl.ds(s * RPT, RPT)],
                    agg_out.at[c, pl.ds(s * RPT, RPT)])
    if with_cnt:
      pltpu.sync_copy(cnt_sh.at[pl.ds(s * RPT, RPT)],
                      cnt_out.at[c, pl.ds(s * RPT, RPT)])

  return pl.kernel(body, out_type=out_type, mesh=mesh,
                   scratch_types=scratch)


_segsum_cnt = _make_segsum(True)
_segsum = _make_segsum(False)


def _dense_body_prelu(agg_ref, cnt_ref, x_ref, wl_ref, b_ref, wr_ref,
                      a_ref, out_ref):
  agg = agg_ref[0, :N, :] + agg_ref[1, :N, :]
  cnt = cnt_ref[0, :N, 0:1] + cnt_ref[1, :N, 0:1]
  mean = agg / jnp.maximum(cnt, 1.0)
  h = lax.dot_general(mean, wl_ref[...], (((1,), (1,)), ((), ())),
                      preferred_element_type=jnp.float32)
  h = h + b_ref[...]
  h = h + lax.dot_general(x_ref[...], wr_ref[...], (((1,), (1,)), ((), ())),
                          preferred_element_type=jnp.float32)
  a = a_ref[...]
  out_ref[...] = jnp.maximum(h, 0.0) + a * jnp.minimum(h, 0.0)


def _dense_body(agg_ref, cnt_ref, x_ref, wl_ref, b_ref, wr_ref, out_ref):
  agg = agg_ref[0, :N, :] + agg_ref[1, :N, :]
  cnt = cnt_ref[0, :N, 0:1] + cnt_ref[1, :N, 0:1]
  mean = agg / jnp.maximum(cnt, 1.0)
  h = lax.dot_general(mean, wl_ref[...], (((1,), (1,)), ((), ())),
                      preferred_element_type=jnp.float32)
  h = h + b_ref[...]
  h = h + lax.dot_general(x_ref[...], wr_ref[...], (((1,), (1,)), ((), ())),
                          preferred_element_type=jnp.float32)
  out_ref[...] = h


_dense1 = pl.pallas_call(
    _dense_body_prelu,
    out_shape=jax.ShapeDtypeStruct((N, D), jnp.float32))
_dense2 = pl.pallas_call(
    _dense_body,
    out_shape=jax.ShapeDtypeStruct((N, D), jnp.float32))


@jax.jit
def kernel(x, edge_index, W1l, b1, W1r, a1, W2l, b2, W2r):
  src = edge_index[0]
  dst = edge_index[1]
  pad = E_PAD - E
  srcp = jnp.concatenate([src, jnp.zeros((pad,), jnp.int32)])
  dstp = jnp.concatenate([dst, jnp.full((pad,), DUMMY, jnp.int32)])
  srcp = srcp.reshape(NW, K, CH)
  dstp = dstp.reshape(NW, K, CH)
  zf = jnp.zeros((CH, D), jnp.float32)
  zc = jnp.zeros((CH, 16), jnp.float32)
  on = jnp.ones((CH, 16), jnp.float32)

  aggp, cntp = _segsum_cnt(x, srcp, dstp, zf, zc, on)
  h = _dense1(aggp, cntp, x, W1l, b1.reshape(1, D), W1r, a1.reshape(1, 1))
  (aggp2,) = _segsum(h, srcp, dstp, zf, zc, on)
  out = _dense2(aggp2, cntp, h, W2l, b2.reshape(1, D), W2r)
  return out


# speedup vs baseline: 3.1541x; 3.1541x over previous
import jax
import jax.numpy as jnp
from jax import lax
from jax.experimental import pallas as pl
from jax.experimental.pallas import tpu as pltpu
from jax.experimental.pallas import tpu_sc as plsc

N = 10000
E = 320000
D = 128

NC = 2
NS = 16
NW = NC * NS
CH = 128
KB = 16
EPW = 10240
K = EPW // CH
E_PAD = NW * EPW
NPAD = 10240
RPT = NPAD // NS
DUMMY = N
NR = NPAD // CH
REG = 8

_mesh = plsc.VectorSubcoreMesh(core_axis_name="c", subcore_axis_name="s",
                               num_cores=NC, num_subcores=NS)


def _segsum_body(x_hbm, src_hbm, dst_hbm, agg_out, src_v, dst_v, rows_v,
                 acc_sh, sem):
  c = lax.axis_index("c")
  s = lax.axis_index("s")
  w = s * NC + c

  zv = jnp.zeros((16,), jnp.float32)

  def zrow(i, carry):
    for kk in range(D // 16):
      rows_v[i, pl.ds(kk * 16, 16)] = zv
    return carry

  lax.fori_loop(0, CH, zrow, 0)
  for r in range(RPT // CH):
    pltpu.sync_copy(rows_v, acc_sh.at[pl.ds(s * RPT + r * CH, CH)])
  plsc.subcore_barrier()

  def blk(b, carry):
    pltpu.sync_copy(src_hbm.at[w, pl.ds(b * KB, KB)], src_v)
    pltpu.sync_copy(dst_hbm.at[w, pl.ds(b * KB, KB)], dst_v)

    def step(j, carry2):
      pltpu.async_copy(x_hbm.at[src_v.at[j]], rows_v, sem).wait()
      pltpu.sync_copy(rows_v, acc_sh.at[dst_v.at[j]], add=True)
      return carry2

    return lax.fori_loop(0, KB, step, carry)

  lax.fori_loop(0, K // KB, blk, 0)
  plsc.subcore_barrier()

  pltpu.sync_copy(acc_sh.at[pl.ds(s * RPT, RPT)],
                  agg_out.at[c, pl.ds(s * RPT, RPT)])


_segsum = pl.kernel(
    _segsum_body,
    out_type=[jax.ShapeDtypeStruct((NC, NPAD, D), jnp.float32)],
    mesh=_mesh,
    scratch_types=[
        pltpu.VMEM((KB, CH), jnp.int32),
        pltpu.VMEM((KB, CH), jnp.int32),
        pltpu.VMEM((CH, D), jnp.float32),
        pltpu.VMEM_SHARED((NPAD, D), jnp.float32),
        pltpu.SemaphoreType.DMA,
    ])


def _count_body(dst_hbm, cnt_out, dst_v, hist):
  c = lax.axis_index("c")
  s = lax.axis_index("s")
  w = s * NC + c
  pltpu.sync_copy(dst_hbm.at[w], dst_v)
  zv = jnp.zeros((16,), jnp.float32)

  def zrow(i, carry):
    for kk in range(CH // 16):
      hist[i, pl.ds(kk * 16, 16)] = zv
    return carry

  lax.fori_loop(0, REG * NR, zrow, 0)

  lane = lax.iota(jnp.int32, 16)
  region = jnp.bitwise_and(lane, REG - 1)
  mlo = lane < 8
  mhi = lane >= 8
  onesv = jnp.ones((16,), jnp.float32)

  def jrow(j, carry):
    for kk in range(CH // 16):
      d = dst_v[j, pl.ds(kk * 16, 16)]
      row = region * NR + lax.shift_right_logical(d, 7)
      col = jnp.bitwise_and(d, 127)
      plsc.addupdate_scatter(hist, [row, col], onesv, mask=mlo)
      plsc.addupdate_scatter(hist, [row, col], onesv, mask=mhi)
    return carry

  lax.fori_loop(0, K, jrow, 0)

  def rrow(i, carry):
    for kk in range(CH // 16):
      acc = hist[i, pl.ds(kk * 16, 16)]
      for r in range(1, REG):
        acc = acc + hist[r * NR + i, pl.ds(kk * 16, 16)]
      hist[i, pl.ds(kk * 16, 16)] = acc
    return carry

  lax.fori_loop(0, NR, rrow, 0)
  pltpu.sync_copy(hist.at[pl.ds(0, NR)], cnt_out.at[w])


_count = pl.kernel(
    _count_body,
    out_type=jax.ShapeDtypeStruct((NW, NR, CH), jnp.float32),
    mesh=_mesh,
    scratch_types=[pltpu.VMEM((K, CH), jnp.int32),
                   pltpu.VMEM((REG * NR, CH), jnp.float32)],
    compiler_params=pltpu.CompilerParams(needs_layout_passes=False))


def _cntsum_body(cnt_ref, out_ref):
  out_ref[...] = jnp.maximum(jnp.sum(cnt_ref[...], axis=0), 1.0)


_cntsum = pl.pallas_call(
    _cntsum_body,
    out_shape=jax.ShapeDtypeStruct((NR, CH), jnp.float32))


def _dense_body_prelu(agg_ref, cnt_ref, x_ref, wl_ref, b_ref, wr_ref,
                      a_ref, out_ref):
  mean = (agg_ref[0, :N, :] + agg_ref[1, :N, :]) / cnt_ref[...]
  h = lax.dot_general(mean, wl_ref[...], (((1,), (1,)), ((), ())),
                      preferred_element_type=jnp.float32)
  h = h + b_ref[...]
  h = h + lax.dot_general(x_ref[...], wr_ref[...], (((1,), (1,)), ((), ())),
                          preferred_element_type=jnp.float32)
  a = a_ref[...]
  out_ref[...] = jnp.maximum(h, 0.0) + a * jnp.minimum(h, 0.0)


def _dense_body(agg_ref, cnt_ref, x_ref, wl_ref, b_ref, wr_ref, out_ref):
  mean = (agg_ref[0, :N, :] + agg_ref[1, :N, :]) / cnt_ref[...]
  h = lax.dot_general(mean, wl_ref[...], (((1,), (1,)), ((), ())),
                      preferred_element_type=jnp.float32)
  h = h + b_ref[...]
  h = h + lax.dot_general(x_ref[...], wr_ref[...], (((1,), (1,)), ((), ())),
                          preferred_element_type=jnp.float32)
  out_ref[...] = h


_dense1 = pl.pallas_call(
    _dense_body_prelu,
    out_shape=jax.ShapeDtypeStruct((N, D), jnp.float32))
_dense2 = pl.pallas_call(
    _dense_body,
    out_shape=jax.ShapeDtypeStruct((N, D), jnp.float32))


@jax.jit
def kernel(x, edge_index, W1l, b1, W1r, a1, W2l, b2, W2r):
  src = edge_index[0]
  dst = edge_index[1]
  pad = E_PAD - E
  srcp = jnp.concatenate([src, jnp.zeros((pad,), jnp.int32)]).reshape(
      NW, K, CH)
  dstp = jnp.concatenate([dst, jnp.full((pad,), DUMMY, jnp.int32)]).reshape(
      NW, K, CH)

  cnt32 = _count(dstp)
  cntc = _cntsum(cnt32).reshape(NPAD, 1)[:N]
  b1r = b1.reshape(1, D)
  b2r = b2.reshape(1, D)

  (aggp,) = _segsum(x, srcp, dstp)
  h = _dense1(aggp, cntc, x, W1l, b1r, W1r, a1.reshape(1, 1))
  (aggp2,) = _segsum(h, srcp, dstp)
  out = _dense2(aggp2, cntc, h, W2l, b2r, W2r)
  return out


# speedup vs baseline: 3.4004x; 1.0781x over previous
import jax
import jax.numpy as jnp
from jax import lax
from jax.experimental import pallas as pl
from jax.experimental.pallas import tpu as pltpu
from jax.experimental.pallas import tpu_sc as plsc

N = 10000
E = 320000
D = 128

NC = 2
NS = 16
NW = NC * NS
CH = 128
KB = 8
EPW = 10240
K = EPW // CH
E_PAD = NW * EPW
NPAD = 10240
RPT = NPAD // NS
DUMMY = N
NR = NPAD // CH
REG = 8

_mesh = plsc.VectorSubcoreMesh(core_axis_name="c", subcore_axis_name="s",
                               num_cores=NC, num_subcores=NS)


def _segsum_body(x_hbm, src_hbm, dst_hbm, agg_out, src_v, dst_v, rows_a,
                 rows_b, acc_sh, sem_a, sem_b):
  c = lax.axis_index("c")
  s = lax.axis_index("s")
  w = s * NC + c

  zv = jnp.zeros((16,), jnp.float32)

  def zrow(i, carry):
    for kk in range(D // 16):
      rows_a[i, pl.ds(kk * 16, 16)] = zv
    return carry

  lax.fori_loop(0, CH, zrow, 0)
  for r in range(RPT // CH):
    pltpu.sync_copy(rows_a, acc_sh.at[pl.ds(s * RPT + r * CH, CH)])
  plsc.subcore_barrier()

  bufs = (rows_a, rows_b)
  sems = (sem_a, sem_b)

  def blk(b, carry):
    pltpu.sync_copy(src_hbm.at[w, pl.ds(b * KB, KB)], src_v)
    pltpu.sync_copy(dst_hbm.at[w, pl.ds(b * KB, KB)], dst_v)
    cps = [None] * KB
    cps[0] = pltpu.async_copy(x_hbm.at[src_v.at[0]], bufs[0], sems[0])
    for jj in range(KB):
      if jj + 1 < KB:
        cps[jj + 1] = pltpu.async_copy(
            x_hbm.at[src_v.at[jj + 1]], bufs[(jj + 1) % 2],
            sems[(jj + 1) % 2])
      cps[jj].wait()
      pltpu.sync_copy(bufs[jj % 2], acc_sh.at[dst_v.at[jj]], add=True)
    return carry

  lax.fori_loop(0, K // KB, blk, 0)
  plsc.subcore_barrier()

  pltpu.sync_copy(acc_sh.at[pl.ds(s * RPT, RPT)],
                  agg_out.at[c, pl.ds(s * RPT, RPT)])


_segsum = pl.kernel(
    _segsum_body,
    out_type=[jax.ShapeDtypeStruct((NC, NPAD, D), jnp.float32)],
    mesh=_mesh,
    scratch_types=[
        pltpu.VMEM((KB, CH), jnp.int32),
        pltpu.VMEM((KB, CH), jnp.int32),
        pltpu.VMEM((CH, D), jnp.float32),
        pltpu.VMEM((CH, D), jnp.float32),
        pltpu.VMEM_SHARED((NPAD, D), jnp.float32),
        pltpu.SemaphoreType.DMA,
        pltpu.SemaphoreType.DMA,
    ])


def _count_body(dst_hbm, cnt_out, dst_v, hist):
  c = lax.axis_index("c")
  s = lax.axis_index("s")
  w = s * NC + c
  pltpu.sync_copy(dst_hbm.at[w], dst_v)
  zv = jnp.zeros((16,), jnp.float32)

  def zrow(i, carry):
    for kk in range(CH // 16):
      hist[i, pl.ds(kk * 16, 16)] = zv
    return carry

  lax.fori_loop(0, REG * NR, zrow, 0)

  lane = lax.iota(jnp.int32, 16)
  region = jnp.bitwise_and(lane, REG - 1)
  mlo = lane < 8
  mhi = lane >= 8
  onesv = jnp.ones((16,), jnp.float32)

  def jrow(j, carry):
    for kk in range(CH // 16):
      d = dst_v[j, pl.ds(kk * 16, 16)]
      row = region * NR + lax.shift_right_logical(d, 7)
      col = jnp.bitwise_and(d, 127)
      plsc.addupdate_scatter(hist, [row, col], onesv, mask=mlo)
      plsc.addupdate_scatter(hist, [row, col], onesv, mask=mhi)
    return carry

  lax.fori_loop(0, K, jrow, 0)

  def rrow(i, carry):
    for kk in range(CH // 16):
      acc = hist[i, pl.ds(kk * 16, 16)]
      for r in range(1, REG):
        acc = acc + hist[r * NR + i, pl.ds(kk * 16, 16)]
      hist[i, pl.ds(kk * 16, 16)] = acc
    return carry

  lax.fori_loop(0, NR, rrow, 0)
  pltpu.sync_copy(hist.at[pl.ds(0, NR)], cnt_out.at[w])


_count = pl.kernel(
    _count_body,
    out_type=jax.ShapeDtypeStruct((NW, NR, CH), jnp.float32),
    mesh=_mesh,
    scratch_types=[pltpu.VMEM((K, CH), jnp.int32),
                   pltpu.VMEM((REG * NR, CH), jnp.float32)],
    compiler_params=pltpu.CompilerParams(needs_layout_passes=False))


def _cntsum_body(cnt_ref, out_ref):
  out_ref[...] = jnp.maximum(jnp.sum(cnt_ref[...], axis=0), 1.0)


_cntsum = pl.pallas_call(
    _cntsum_body,
    out_shape=jax.ShapeDtypeStruct((NR, CH), jnp.float32))


def _dense_body_prelu(agg_ref, cnt_ref, x_ref, wl_ref, b_ref, wr_ref,
                      a_ref, out_ref):
  mean = (agg_ref[0, :N, :] + agg_ref[1, :N, :]) / cnt_ref[...]
  h = lax.dot_general(mean, wl_ref[...], (((1,), (1,)), ((), ())),
                      preferred_element_type=jnp.float32)
  h = h + b_ref[...]
  h = h + lax.dot_general(x_ref[...], wr_ref[...], (((1,), (1,)), ((), ())),
                          preferred_element_type=jnp.float32)
  a = a_ref[...]
  out_ref[...] = jnp.maximum(h, 0.0) + a * jnp.minimum(h, 0.0)


def _dense_body(agg_ref, cnt_ref, x_ref, wl_ref, b_ref, wr_ref, out_ref):
  mean = (agg_ref[0, :N, :] + agg_ref[1, :N, :]) / cnt_ref[...]
  h = lax.dot_general(mean, wl_ref[...], (((1,), (1,)), ((), ())),
                      preferred_element_type=jnp.float32)
  h = h + b_ref[...]
  h = h + lax.dot_general(x_ref[...], wr_ref[...], (((1,), (1,)), ((), ())),
                          preferred_element_type=jnp.float32)
  out_ref[...] = h


_dense1 = pl.pallas_call(
    _dense_body_prelu,
    out_shape=jax.ShapeDtypeStruct((N, D), jnp.float32))
_dense2 = pl.pallas_call(
    _dense_body,
    out_shape=jax.ShapeDtypeStruct((N, D), jnp.float32))


@jax.jit
def kernel(x, edge_index, W1l, b1, W1r, a1, W2l, b2, W2r):
  src = edge_index[0]
  dst = edge_index[1]
  pad = E_PAD - E
  srcp = jnp.concatenate([src, jnp.zeros((pad,), jnp.int32)]).reshape(
      NW, K, CH)
  dstp = jnp.concatenate([dst, jnp.full((pad,), DUMMY, jnp.int32)]).reshape(
      NW, K, CH)

  cnt32 = _count(dstp)
  cntc = _cntsum(cnt32).reshape(NPAD, 1)[:N]
  b1r = b1.reshape(1, D)
  b2r = b2.reshape(1, D)

  (aggp,) = _segsum(x, srcp, dstp)
  h = _dense1(aggp, cntc, x, W1l, b1r, W1r, a1.reshape(1, 1))
  (aggp2,) = _segsum(h, srcp, dstp)
  out = _dense2(aggp2, cntc, h, W2l, b2r, W2r)
  return out


# speedup vs baseline: 3.4075x; 1.0021x over previous
import jax
import jax.numpy as jnp
from jax import lax
from jax.experimental import pallas as pl
from jax.experimental.pallas import tpu as pltpu
from jax.experimental.pallas import tpu_sc as plsc

N = 10000
E = 320000
D = 128

NC = 2
NS = 16
NW = NC * NS
CH = 128
KB = 8
EPW = 10240
K = EPW // CH
E_PAD = NW * EPW
NPAD = 10240
RPT = NPAD // NS
DUMMY = N
NR = NPAD // CH
REG = 8

_mesh = plsc.VectorSubcoreMesh(core_axis_name="c", subcore_axis_name="s",
                               num_cores=NC, num_subcores=NS)


def _segsum_body(x_hbm, src_hbm, dst_hbm, agg_out, src_v, dst_v, rows_a,
                 rows_b, acc_sh, sem_a, sem_b):
  c = lax.axis_index("c")
  s = lax.axis_index("s")
  w = s * NC + c

  zv = jnp.zeros((16,), jnp.float32)

  def zrow(i, carry):
    for kk in range(D // 16):
      rows_a[i, pl.ds(kk * 16, 16)] = zv
    return carry

  lax.fori_loop(0, CH, zrow, 0)
  for r in range(RPT // CH):
    pltpu.sync_copy(rows_a, acc_sh.at[pl.ds(s * RPT + r * CH, CH)])
  plsc.subcore_barrier()

  bufs = (rows_a, rows_b)
  sems = (sem_a, sem_b)

  def blk(b, carry):
    pltpu.sync_copy(src_hbm.at[w, pl.ds(b * KB, KB)], src_v)
    pltpu.sync_copy(dst_hbm.at[w, pl.ds(b * KB, KB)], dst_v)
    cps = [None] * KB
    cps[0] = pltpu.async_copy(x_hbm.at[src_v.at[0]], bufs[0], sems[0])
    for jj in range(KB):
      if jj + 1 < KB:
        cps[jj + 1] = pltpu.async_copy(
            x_hbm.at[src_v.at[jj + 1]], bufs[(jj + 1) % 2],
            sems[(jj + 1) % 2])
      cps[jj].wait()
      pltpu.sync_copy(bufs[jj % 2], acc_sh.at[dst_v.at[jj]], add=True)
    return carry

  lax.fori_loop(0, K // KB, blk, 0)
  plsc.subcore_barrier()

  pltpu.sync_copy(acc_sh.at[pl.ds(s * RPT, RPT)],
                  agg_out.at[c, pl.ds(s * RPT, RPT)])


_segsum = pl.kernel(
    _segsum_body,
    out_type=[jax.ShapeDtypeStruct((NC, NPAD, D), jnp.float32)],
    mesh=_mesh,
    scratch_types=[
        pltpu.VMEM((KB, CH), jnp.int32),
        pltpu.VMEM((KB, CH), jnp.int32),
        pltpu.VMEM((CH, D), jnp.float32),
        pltpu.VMEM((CH, D), jnp.float32),
        pltpu.VMEM_SHARED((NPAD, D), jnp.float32),
        pltpu.SemaphoreType.DMA,
        pltpu.SemaphoreType.DMA,
    ])


def _count_body(dst_hbm, cnt_out, dst_v, hist):
  c = lax.axis_index("c")
  s = lax.axis_index("s")
  w = s * NC + c
  pltpu.sync_copy(dst_hbm.at[w], dst_v)
  zv = jnp.zeros((16,), jnp.float32)

  def zrow(i, carry):
    for kk in range(CH // 16):
      hist[i, pl.ds(kk * 16, 16)] = zv
    return carry

  lax.fori_loop(0, REG * NR, zrow, 0)

  lane = lax.iota(jnp.int32, 16)
  region = jnp.bitwise_and(lane, REG - 1)
  mlo = lane < 8
  mhi = lane >= 8
  onesv = jnp.ones((16,), jnp.float32)

  def jrow(j, carry):
    for kk in range(CH // 16):
      d = dst_v[j, pl.ds(kk * 16, 16)]
      row = region * NR + lax.shift_right_logical(d, 7)
      col = jnp.bitwise_and(d, 127)
      plsc.addupdate_scatter(hist, [row, col], onesv, mask=mlo)
      plsc.addupdate_scatter(hist, [row, col], onesv, mask=mhi)
    return carry

  lax.fori_loop(0, K, jrow, 0)

  def rrow(i, carry):
    for kk in range(CH // 16):
      acc = hist[i, pl.ds(kk * 16, 16)]
      for r in range(1, REG):
        acc = acc + hist[r * NR + i, pl.ds(kk * 16, 16)]
      hist[i, pl.ds(kk * 16, 16)] = acc
    return carry

  lax.fori_loop(0, NR, rrow, 0)
  pltpu.sync_copy(hist.at[pl.ds(0, NR)], cnt_out.at[w])


_count = pl.kernel(
    _count_body,
    out_type=jax.ShapeDtypeStruct((NW, NR, CH), jnp.float32),
    mesh=_mesh,
    scratch_types=[pltpu.VMEM((K, CH), jnp.int32),
                   pltpu.VMEM((REG * NR, CH), jnp.float32)],
    compiler_params=pltpu.CompilerParams(needs_layout_passes=False))


def _cntsum_body(cnt_ref, out_ref):
  out_ref[...] = jnp.maximum(jnp.sum(cnt_ref[...], axis=0), 1.0)


_cntsum = pl.pallas_call(
    _cntsum_body,
    out_shape=jax.ShapeDtypeStruct((NR, CH), jnp.float32))


def _dense_body_prelu(agg_ref, cnt_ref, x_ref, wl_ref, b_ref, wr_ref,
                      a_ref, out_ref):
  mean = (agg_ref[0, :N, :] + agg_ref[1, :N, :]) / cnt_ref[...]
  h = lax.dot_general(mean, wl_ref[...], (((1,), (1,)), ((), ())),
                      preferred_element_type=jnp.float32)
  h = h + b_ref[...]
  h = h + lax.dot_general(x_ref[...], wr_ref[...], (((1,), (1,)), ((), ())),
                          preferred_element_type=jnp.float32)
  a = a_ref[...]
  out_ref[...] = jnp.maximum(h, 0.0) + a * jnp.minimum(h, 0.0)


def _dense_body(agg_ref, cnt_ref, x_ref, wl_ref, b_ref, wr_ref, out_ref):
  mean = (agg_ref[0, :N, :] + agg_ref[1, :N, :]) / cnt_ref[...]
  h = lax.dot_general(mean, wl_ref[...], (((1,), (1,)), ((), ())),
                      preferred_element_type=jnp.float32)
  h = h + b_ref[...]
  h = h + lax.dot_general(x_ref[...], wr_ref[...], (((1,), (1,)), ((), ())),
                          preferred_element_type=jnp.float32)
  out_ref[...] = h


_dense1 = pl.pallas_call(
    _dense_body_prelu,
    out_shape=jax.ShapeDtypeStruct((N, D), jnp.float32))
_dense2 = pl.pallas_call(
    _dense_body,
    out_shape=jax.ShapeDtypeStruct((N, D), jnp.float32))


@jax.jit
def kernel(x, edge_index, W1l, b1, W1r, a1, W2l, b2, W2r):
  src = edge_index[0]
  dst = edge_index[1]
  pad = E_PAD - E
  srcp = jnp.concatenate([src, jnp.zeros((pad,), jnp.int32)]).reshape(
      NW, K, CH)
  dpad = DUMMY + jnp.arange(pad, dtype=jnp.int32) % (NPAD - N)
  dstp = jnp.concatenate([dst, dpad]).reshape(NW, K, CH)

  cnt32 = _count(dstp)
  cntc = _cntsum(cnt32).reshape(NPAD, 1)[:N]
  b1r = b1.reshape(1, D)
  b2r = b2.reshape(1, D)

  (aggp,) = _segsum(x, srcp, dstp)
  h = _dense1(aggp, cntc, x, W1l, b1r, W1r, a1.reshape(1, 1))
  (aggp2,) = _segsum(h, srcp, dstp)
  out = _dense2(aggp2, cntc, h, W2l, b2r, W2r)
  return out


# speedup vs baseline: 4.1505x; 1.2181x over previous
import jax
import jax.numpy as jnp
from jax import lax
from jax.experimental import pallas as pl
from jax.experimental.pallas import tpu as pltpu
from jax.experimental.pallas import tpu_sc as plsc

N = 10000
E = 320000
D = 128

NC = 2
NS = 16
NW = NC * NS
CH = 128
KB = 8
EPW = 10240
K = EPW // CH
E_PAD = NW * EPW
NBLK = E_PAD // (KB * CH)
B0 = 15
B1 = 5
NPAD = 10240
RPT = NPAD // NS
DUMMY = N
NR = NPAD // CH
REG = 8

_mesh = plsc.VectorSubcoreMesh(core_axis_name="c", subcore_axis_name="s",
                               num_cores=NC, num_subcores=NS)


def _segsum_body(x_hbm, src_hbm, dst_hbm, agg_out, src_v, dst_v, rows_a,
                 rows_b, acc_sh, sem_a, sem_b):
  c = lax.axis_index("c")
  s = lax.axis_index("s")
  w = s * NC + c

  zv = jnp.zeros((16,), jnp.float32)

  def zrow(i, carry):
    for kk in range(D // 16):
      rows_a[i, pl.ds(kk * 16, 16)] = zv
    return carry

  lax.fori_loop(0, CH, zrow, 0)
  for r in range(RPT // CH):
    pltpu.sync_copy(rows_a, acc_sh.at[pl.ds(s * RPT + r * CH, CH)])
  plsc.subcore_barrier()

  bufs = (rows_a, rows_b)
  sems = (sem_a, sem_b)
  nblk = B0 + c * (B1 - B0)
  base = (1 - c) * (s * B0) + c * (NS * B0 + s * B1)

  def blk(b, carry):
    pltpu.sync_copy(src_hbm.at[base + b], src_v)
    pltpu.sync_copy(dst_hbm.at[base + b], dst_v)
    cps = [None] * KB
    cps[0] = pltpu.async_copy(x_hbm.at[src_v.at[0]], bufs[0], sems[0])
    for jj in range(KB):
      if jj + 1 < KB:
        cps[jj + 1] = pltpu.async_copy(
            x_hbm.at[src_v.at[jj + 1]], bufs[(jj + 1) % 2],
            sems[(jj + 1) % 2])
      cps[jj].wait()
      pltpu.sync_copy(bufs[jj % 2], acc_sh.at[dst_v.at[jj]], add=True)
    return carry

  lax.fori_loop(0, nblk, blk, 0)
  plsc.subcore_barrier()

  pltpu.sync_copy(acc_sh.at[pl.ds(s * RPT, RPT)],
                  agg_out.at[c, pl.ds(s * RPT, RPT)])


_segsum = pl.kernel(
    _segsum_body,
    out_type=[jax.ShapeDtypeStruct((NC, NPAD, D), jnp.float32)],
    mesh=_mesh,
    scratch_types=[
        pltpu.VMEM((KB, CH), jnp.int32),
        pltpu.VMEM((KB, CH), jnp.int32),
        pltpu.VMEM((CH, D), jnp.float32),
        pltpu.VMEM((CH, D), jnp.float32),
        pltpu.VMEM_SHARED((NPAD, D), jnp.float32),
        pltpu.SemaphoreType.DMA,
        pltpu.SemaphoreType.DMA,
    ])


def _count_body(dst_hbm, cnt_out, dst_v, hist):
  c = lax.axis_index("c")
  s = lax.axis_index("s")
  w = s * NC + c
  pltpu.sync_copy(dst_hbm.at[w], dst_v)
  zv = jnp.zeros((16,), jnp.float32)

  def zrow(i, carry):
    for kk in range(CH // 16):
      hist[i, pl.ds(kk * 16, 16)] = zv
    return carry

  lax.fori_loop(0, REG * NR, zrow, 0)

  lane = lax.iota(jnp.int32, 16)
  region = jnp.bitwise_and(lane, REG - 1)
  mlo = lane < 8
  mhi = lane >= 8
  onesv = jnp.ones((16,), jnp.float32)

  def jrow(j, carry):
    for kk in range(CH // 16):
      d = dst_v[j, pl.ds(kk * 16, 16)]
      row = region * NR + lax.shift_right_logical(d, 7)
      col = jnp.bitwise_and(d, 127)
      plsc.addupdate_scatter(hist, [row, col], onesv, mask=mlo)
      plsc.addupdate_scatter(hist, [row, col], onesv, mask=mhi)
    return carry

  lax.fori_loop(0, K, jrow, 0)

  def rrow(i, carry):
    for kk in range(CH // 16):
      acc = hist[i, pl.ds(kk * 16, 16)]
      for r in range(1, REG):
        acc = acc + hist[r * NR + i, pl.ds(kk * 16, 16)]
      hist[i, pl.ds(kk * 16, 16)] = acc
    return carry

  lax.fori_loop(0, NR, rrow, 0)
  pltpu.sync_copy(hist.at[pl.ds(0, NR)], cnt_out.at[w])


_count = pl.kernel(
    _count_body,
    out_type=jax.ShapeDtypeStruct((NW, NR, CH), jnp.float32),
    mesh=_mesh,
    scratch_types=[pltpu.VMEM((K, CH), jnp.int32),
                   pltpu.VMEM((REG * NR, CH), jnp.float32)],
    compiler_params=pltpu.CompilerParams(needs_layout_passes=False))


def _cntsum_body(cnt_ref, out_ref):
  out_ref[...] = jnp.maximum(jnp.sum(cnt_ref[...], axis=0), 1.0)


_cntsum = pl.pallas_call(
    _cntsum_body,
    out_shape=jax.ShapeDtypeStruct((NR, CH), jnp.float32))


def _dense_body_prelu(agg_ref, cnt_ref, x_ref, wl_ref, b_ref, wr_ref,
                      a_ref, out_ref):
  mean = (agg_ref[0, :N, :] + agg_ref[1, :N, :]) / cnt_ref[...]
  h = lax.dot_general(mean, wl_ref[...], (((1,), (1,)), ((), ())),
                      preferred_element_type=jnp.float32)
  h = h + b_ref[...]
  h = h + lax.dot_general(x_ref[...], wr_ref[...], (((1,), (1,)), ((), ())),
                          preferred_element_type=jnp.float32)
  a = a_ref[...]
  out_ref[...] = jnp.maximum(h, 0.0) + a * jnp.minimum(h, 0.0)


def _dense_body(agg_ref, cnt_ref, x_ref, wl_ref, b_ref, wr_ref, out_ref):
  mean = (agg_ref[0, :N, :] + agg_ref[1, :N, :]) / cnt_ref[...]
  h = lax.dot_general(mean, wl_ref[...], (((1,), (1,)), ((), ())),
                      preferred_element_type=jnp.float32)
  h = h + b_ref[...]
  h = h + lax.dot_general(x_ref[...], wr_ref[...], (((1,), (1,)), ((), ())),
                          preferred_element_type=jnp.float32)
  out_ref[...] = h


_dense1 = pl.pallas_call(
    _dense_body_prelu,
    out_shape=jax.ShapeDtypeStruct((N, D), jnp.float32))
_dense2 = pl.pallas_call(
    _dense_body,
    out_shape=jax.ShapeDtypeStruct((N, D), jnp.float32))


@jax.jit
def kernel(x, edge_index, W1l, b1, W1r, a1, W2l, b2, W2r):
  src = edge_index[0]
  dst = edge_index[1]
  pad = E_PAD - E
  srcf = jnp.concatenate([src, jnp.zeros((pad,), jnp.int32)])
  dpad = DUMMY + jnp.arange(pad, dtype=jnp.int32) % (NPAD - N)
  dstf = jnp.concatenate([dst, dpad])
  srcp = srcf.reshape(NBLK, KB, CH)
  dstp = dstf.reshape(NBLK, KB, CH)

  cnt32 = _count(dstf.reshape(NW, K, CH))
  cntc = _cntsum(cnt32).reshape(NPAD, 1)[:N]
  b1r = b1.reshape(1, D)
  b2r = b2.reshape(1, D)

  (aggp,) = _segsum(x, srcp, dstp)
  h = _dense1(aggp, cntc, x, W1l, b1r, W1r, a1.reshape(1, 1))
  (aggp2,) = _segsum(h, srcp, dstp)
  out = _dense2(aggp2, cntc, h, W2l, b2r, W2r)
  return out
